# baseline (speedup 1.0000x reference)
import jax
import jax.numpy as jnp
from jax.experimental import pallas as pl
from jax.experimental.pallas import tpu as pltpu

B, C, H, W, D = 4, 256, 32, 32, 32
BC = 32
HB = BC // 2
NC = C // BC
NCHUNK = B * NC
G = (W * D) // 128


def _pool_sum(b_ref):
    t0 = b_ref[:, 0::2, :, :]
    t1 = b_ref[:, 1::2, :, :]
    m1 = jnp.maximum(t0, t1)
    s1 = t0 + t1
    m2 = jnp.maximum(m1, pltpu.roll(m1, 127, axis=3))
    m3 = jnp.maximum(m2, pltpu.roll(m2, 96, axis=3))
    l = jax.lax.broadcasted_iota(jnp.int32, (G, 128), 1)
    valid = ((l % 2) == 0) & ((l % 64) < 32)
    val = jnp.where(valid, m3, 0.0) + s1 * 0.125
    return jnp.sum(val, axis=(1, 2, 3))


def _copies(x_hbm, y_hbm, z_hbm, xb, yb, zb, sem, k, slot):
    b = k // NC
    j = k % NC
    descs = []
    pris = (0, 0, 1, 1, 0, 1)
    for ci, (hbm, buf) in enumerate(((x_hbm, xb), (y_hbm, yb), (z_hbm, zb))):
        for hh in range(2):
            src = hbm.at[b, pl.ds(j * BC + hh * HB, HB)]
            dst = buf.at[slot, pl.ds(hh * HB, HB)]
            si = 2 * ci + hh
            descs.append((pltpu.make_async_copy(src, dst, sem.at[si, slot]),
                          pris[si]))
    return descs


def _body(x_hbm, y_hbm, z_hbm, o_ref, xb, yb, zb, sem):
    for d, pri in _copies(x_hbm, y_hbm, z_hbm, xb, yb, zb, sem, 0, 0):
        d.start(priority=pri)

    def step(k, carry):
        slot = jax.lax.rem(k, 2)

        @pl.when(k < NCHUNK - 1)
        def _():
            nxt = _copies(x_hbm, y_hbm, z_hbm, xb, yb, zb, sem,
                          k + 1, 1 - slot)
            for d, pri in nxt:
                d.start(priority=pri)

        for d, _ in _copies(x_hbm, y_hbm, z_hbm, xb, yb, zb, sem, k, slot):
            d.wait()

        sx = _pool_sum(xb.at[slot])
        sy = _pool_sum(yb.at[slot])
        sz = _pool_sum(zb.at[slot])
        sxy = sx + sy
        syz = sy + sz
        sxz = sx + sz
        pooled = sxy * sxy + syz * syz + sxz * sxz
        o_ref[pl.ds(k // NC, 1), pl.ds(k % NC, 1), :] = (
            pooled.reshape(1, 1, BC))
        return carry

    jax.lax.fori_loop(0, NCHUNK, step, 0)

    rows = o_ref[...]
    n2 = jnp.sum(rows * rows, axis=(1, 2), keepdims=True)
    inv = 1.0 / jnp.maximum(jnp.sqrt(n2), 1e-12)
    o_ref[...] = rows * inv


def kernel(x, y, z):
    xr = x.reshape(B, C, H, G, 128)
    yr = y.reshape(B, C, H, G, 128)
    zr = z.reshape(B, C, H, G, 128)
    any_spec = pl.BlockSpec(memory_space=pl.ANY)
    out = pl.pallas_call(
        _body,
        in_specs=[any_spec, any_spec, any_spec],
        out_specs=pl.BlockSpec(memory_space=pltpu.MemorySpace.VMEM),
        out_shape=jax.ShapeDtypeStruct((B, NC, BC), jnp.float32),
        scratch_shapes=[
            pltpu.VMEM((2, BC, H, G, 128), jnp.float32),
            pltpu.VMEM((2, BC, H, G, 128), jnp.float32),
            pltpu.VMEM((2, BC, H, G, 128), jnp.float32),
            pltpu.SemaphoreType.DMA((6, 2)),
        ],
        compiler_params=pltpu.CompilerParams(
            vmem_limit_bytes=56 * 1024 * 1024,
        ),
    )(xr, yr, zr)
    return out.reshape(B, C)


# speedup vs baseline: 1.0061x; 1.0061x over previous
import jax
import jax.numpy as jnp
from jax.experimental import pallas as pl
from jax.experimental.pallas import tpu as pltpu

B, C, H, W, D = 4, 256, 32, 32, 32
BC = 32
NC = C // BC
G = (W * D) // 128


def _pool_sum(a_ref):
    t0 = a_ref[:, :, 0::2, :, :]
    t1 = a_ref[:, :, 1::2, :, :]
    m1 = jnp.maximum(t0, t1)
    s1 = t0 + t1
    m2 = jnp.maximum(m1, pltpu.roll(m1, 127, axis=4))
    m3 = jnp.maximum(m2, pltpu.roll(m2, 96, axis=4))
    l = jax.lax.broadcasted_iota(jnp.int32, (G, 128), 1)
    valid = ((l % 2) == 0) & ((l % 64) < 32)
    val = jnp.where(valid, m3, 0.0) + s1 * 0.125
    return jnp.sum(val, axis=(2, 3, 4))


def _body(x_ref, y_ref, z_ref, o_ref):
    j = pl.program_id(1)
    sx = _pool_sum(x_ref)
    sy = _pool_sum(y_ref)
    sz = _pool_sum(z_ref)
    sxy = sx + sy
    syz = sy + sz
    sxz = sx + sz
    pooled = sxy * sxy + syz * syz + sxz * sxz
    o_ref[:, pl.ds(j, 1), :] = pooled.reshape(1, 1, BC)

    @pl.when(j == NC - 1)
    def _():
        row = o_ref[...]
        inv = 1.0 / jnp.maximum(jnp.sqrt(jnp.sum(row * row)), 1e-12)
        o_ref[...] = row * inv


def kernel(x, y, z):
    xr = x.reshape(B, C, H, G, 128)
    yr = y.reshape(B, C, H, G, 128)
    zr = z.reshape(B, C, H, G, 128)
    spec = pl.BlockSpec((1, BC, H, G, 128), lambda b, j: (b, j, 0, 0, 0))
    out = pl.pallas_call(
        _body,
        grid=(B, NC),
        in_specs=[spec, spec, spec],
        out_specs=pl.BlockSpec((1, NC, BC), lambda b, j: (b, 0, 0)),
        out_shape=jax.ShapeDtypeStruct((B, NC, BC), jnp.float32),
        compiler_params=pltpu.CompilerParams(
            dimension_semantics=("parallel", "arbitrary"),
            vmem_limit_bytes=56 * 1024 * 1024,
        ),
    )(xr, yr, zr)
    return out.reshape(B, C)
